# baseline (speedup 1.0000x reference)
import functools

import jax
import jax.numpy as jnp
from jax import lax
from jax.experimental import pallas as pl
from jax.experimental.pallas import tpu as pltpu
from jax.experimental.pallas import tpu_sc as plsc

N_TOKENS = 8192
D_MODEL = 4096
NUM_GROUPS = 16
BLOCK_N = 512

_LANES = 16
_NUM_WORKERS = 32
SC_TOKENS = 2048
_TC_TOKENS = N_TOKENS - SC_TOKENS
_TOK0 = _TC_TOKENS
_T_W = SC_TOKENS // _NUM_WORKERS
_SC_CHUNK = 4
_UNROLL = 8


def _fused_kernel(x_ref, g_ref, w_ref, b_ref, o_ref):
    xb = x_ref[...]
    scores = lax.dot_general(
        xb, w_ref[...], (((1,), (1,)), ((), ())),
        preferred_element_type=jnp.float32)
    gid = g_ref[...]
    cols = lax.broadcasted_iota(jnp.int32, (xb.shape[0], NUM_GROUPS), 1)
    onehot = (cols == gid).astype(jnp.float32)
    o_ref[...] = jnp.sum((scores + b_ref[...]) * onehot, axis=1, keepdims=True)


def _tc_part(x_tc, g_tc, w2, b2):
    grid = _TC_TOKENS // BLOCK_N
    return pl.pallas_call(
        _fused_kernel,
        grid=(grid,),
        in_specs=[
            pl.BlockSpec((BLOCK_N, D_MODEL), lambda i: (i, 0)),
            pl.BlockSpec((BLOCK_N, 1), lambda i: (i, 0)),
            pl.BlockSpec((NUM_GROUPS, D_MODEL), lambda i: (0, 0)),
            pl.BlockSpec((1, NUM_GROUPS), lambda i: (0, 0)),
        ],
        out_specs=pl.BlockSpec((BLOCK_N, 1), lambda i: (i, 0)),
        out_shape=jax.ShapeDtypeStruct((_TC_TOKENS, 1), jnp.float32),
    )(x_tc, g_tc, w2, b2)


def _sc_part(x, group_sc, group_pad, w2, b_flat):
    mesh = plsc.VectorSubcoreMesh(core_axis_name="c", subcore_axis_name="s")
    steps = D_MODEL // (_LANES * _UNROLL)

    n_chunks = _T_W // _SC_CHUNK

    @functools.partial(
        pl.kernel, mesh=mesh,
        out_type=jax.ShapeDtypeStruct((SC_TOKENS,), jnp.float32),
        scratch_types=[
            pltpu.VMEM((2, _SC_CHUNK, D_MODEL), jnp.float32),
            pltpu.VMEM((2, _SC_CHUNK, D_MODEL), jnp.float32),
            pltpu.VMEM((8 * n_chunks,), jnp.int32),
            pltpu.VMEM((_T_W,), jnp.int32),
            pltpu.VMEM((_T_W,), jnp.float32),
            pltpu.VMEM((_T_W,), jnp.float32),
            pltpu.VMEM((2 * _LANES,), jnp.float32),
            pltpu.SemaphoreType.DMA,
            pltpu.SemaphoreType.DMA,
            pltpu.SemaphoreType.DMA,
        ],
    )
    def dot_k(x_hbm, g_hbm, gp_hbm, w_hbm, b_hbm, out_hbm,
              x_v, w_v, gp_v, g_v, b_v, y_v, rot_v, sem0, sem1, semb):
        wid = lax.axis_index("s") * 2 + lax.axis_index("c")
        base = wid * _T_W
        lane = lax.broadcasted_iota(jnp.int32, (_LANES,), 0)
        sems = (sem0, sem1)
        pltpu.sync_copy(gp_hbm.at[pl.ds(wid * 8 * n_chunks, 8 * n_chunks)],
                        gp_v)
        pltpu.sync_copy(g_hbm.at[pl.ds(base, _T_W)], g_v)
        hb = pltpu.async_copy(b_hbm.at[g_v], b_v, semb)

        def start_chunk(c):
            slot = c % 2
            tok = base + c * _SC_CHUNK
            hx = pltpu.async_copy(
                x_hbm.at[pl.ds(_TOK0 + tok, _SC_CHUNK)], x_v.at[slot],
                sems[slot])
            hw = pltpu.async_copy(
                w_hbm.at[gp_v.at[pl.ds(c * 8, _SC_CHUNK)]],
                w_v.at[slot], sems[slot])
            return hx, hw

        pending = start_chunk(0)
        hb.wait()
        acc16 = jnp.zeros((_LANES,), jnp.float32)
        for c in range(n_chunks):
            slot = c % 2
            pending[0].wait()
            pending[1].wait()
            if c + 1 < n_chunks:
                pending = start_chunk(c + 1)
            for t in range(_SC_CHUNK):
                ti = c * _SC_CHUNK + t

                z = jnp.zeros((_LANES,), jnp.float32)

                @plsc.parallel_loop(0, steps, unroll=4, carry=(z, z, z, z))
                def accs(j, carry):
                    a0, a1, a2, a3 = carry
                    o = j * (_LANES * _UNROLL)
                    for k in range(_UNROLL):
                        off = o + k * _LANES
                        prod = (x_v[slot, t, pl.ds(off, _LANES)]
                                * w_v[slot, t, pl.ds(off, _LANES)])
                        if k % 4 == 0:
                            a0 = a0 + prod
                        elif k % 4 == 1:
                            a1 = a1 + prod
                        elif k % 4 == 2:
                            a2 = a2 + prod
                        else:
                            a3 = a3 + prod
                    return (a0, a1, a2, a3)

                v = (accs[0] + accs[1]) + (accs[2] + accs[3])
                for sh in (8, 4, 2, 1):
                    rot_v[pl.ds(0, _LANES)] = v
                    rot_v[pl.ds(_LANES, _LANES)] = v
                    v = v + rot_v[pl.ds(sh, _LANES)]
                acc16 = acc16 + jnp.where(lane == ti % _LANES, v, 0.0)
                if ti % _LANES == _LANES - 1:
                    blk = (ti // _LANES) * _LANES
                    y_v[pl.ds(blk, _LANES)] = (
                        acc16 + b_v[pl.ds(blk, _LANES)])
                    acc16 = jnp.zeros((_LANES,), jnp.float32)
        pltpu.sync_copy(y_v, out_hbm.at[pl.ds(base, _T_W)])

    return dot_k(x, group_sc, group_pad, w2, b_flat)


def kernel(x, group, W, b):
    g1 = group.astype(jnp.int32)
    w2 = W.reshape(NUM_GROUPS, D_MODEL)
    b2 = b.reshape(1, NUM_GROUPS)
    g_sc = g1[_TC_TOKENS:]
    g_pad = jnp.zeros((SC_TOKENS // _SC_CHUNK, 8), jnp.int32)
    g_pad = g_pad.at[:, :_SC_CHUNK].set(g_sc.reshape(-1, _SC_CHUNK))
    y_tc = _tc_part(x, g1.reshape(-1, 1), w2, b2)
    y_sc = _sc_part(x, g_sc, g_pad.reshape(-1), w2, b.reshape(-1))
    return jnp.concatenate([y_tc, y_sc.reshape(SC_TOKENS, 1)], axis=0)


# speedup vs baseline: 1.1726x; 1.1726x over previous
import functools

import jax
import jax.numpy as jnp
from jax import lax
from jax.experimental import pallas as pl
from jax.experimental.pallas import tpu as pltpu
from jax.experimental.pallas import tpu_sc as plsc

N_TOKENS = 8192
D_MODEL = 4096
NUM_GROUPS = 16
BLOCK_N = 512

_LANES = 16
_NUM_WORKERS = 32
SC_TOKENS = 1024
_TC_TOKENS = N_TOKENS - SC_TOKENS
_TOK0 = _TC_TOKENS
_T_W = SC_TOKENS // _NUM_WORKERS
_SC_CHUNK = 4
_UNROLL = 8


def _fused_kernel(x_ref, g_ref, w_ref, b_ref, o_ref):
    xb = x_ref[...]
    scores = lax.dot_general(
        xb, w_ref[...], (((1,), (1,)), ((), ())),
        preferred_element_type=jnp.float32)
    gid = g_ref[...]
    cols = lax.broadcasted_iota(jnp.int32, (xb.shape[0], NUM_GROUPS), 1)
    onehot = (cols == gid).astype(jnp.float32)
    o_ref[...] = jnp.sum((scores + b_ref[...]) * onehot, axis=1, keepdims=True)


def _tc_part(x_tc, g_tc, w2, b2):
    grid = _TC_TOKENS // BLOCK_N
    return pl.pallas_call(
        _fused_kernel,
        grid=(grid,),
        in_specs=[
            pl.BlockSpec((BLOCK_N, D_MODEL), lambda i: (i, 0)),
            pl.BlockSpec((BLOCK_N, 1), lambda i: (i, 0)),
            pl.BlockSpec((NUM_GROUPS, D_MODEL), lambda i: (0, 0)),
            pl.BlockSpec((1, NUM_GROUPS), lambda i: (0, 0)),
        ],
        out_specs=pl.BlockSpec((BLOCK_N, 1), lambda i: (i, 0)),
        out_shape=jax.ShapeDtypeStruct((_TC_TOKENS, 1), jnp.float32),
    )(x_tc, g_tc, w2, b2)


def _sc_part(x, group_sc, group_pad, w2, b_flat):
    mesh = plsc.VectorSubcoreMesh(core_axis_name="c", subcore_axis_name="s")
    steps = D_MODEL // (_LANES * _UNROLL)

    n_chunks = _T_W // _SC_CHUNK

    @functools.partial(
        pl.kernel, mesh=mesh,
        out_type=jax.ShapeDtypeStruct((SC_TOKENS,), jnp.float32),
        scratch_types=[
            pltpu.VMEM((2, _SC_CHUNK, D_MODEL), jnp.float32),
            pltpu.VMEM((2, _SC_CHUNK, D_MODEL), jnp.float32),
            pltpu.VMEM((8 * n_chunks,), jnp.int32),
            pltpu.VMEM((_T_W,), jnp.int32),
            pltpu.VMEM((_T_W,), jnp.float32),
            pltpu.VMEM((_T_W,), jnp.float32),
            pltpu.VMEM((2 * _LANES,), jnp.float32),
            pltpu.SemaphoreType.DMA,
            pltpu.SemaphoreType.DMA,
            pltpu.SemaphoreType.DMA,
        ],
    )
    def dot_k(x_hbm, g_hbm, gp_hbm, w_hbm, b_hbm, out_hbm,
              x_v, w_v, gp_v, g_v, b_v, y_v, rot_v, sem0, sem1, semb):
        wid = lax.axis_index("s") * 2 + lax.axis_index("c")
        base = wid * _T_W
        lane = lax.broadcasted_iota(jnp.int32, (_LANES,), 0)
        sems = (sem0, sem1)
        pltpu.sync_copy(gp_hbm.at[pl.ds(wid * 8 * n_chunks, 8 * n_chunks)],
                        gp_v)
        pltpu.sync_copy(g_hbm.at[pl.ds(base, _T_W)], g_v)
        hb = pltpu.async_copy(b_hbm.at[g_v], b_v, semb)

        def start_chunk(c):
            slot = c % 2
            tok = base + c * _SC_CHUNK
            hx = pltpu.async_copy(
                x_hbm.at[pl.ds(_TOK0 + tok, _SC_CHUNK)], x_v.at[slot],
                sems[slot])
            hw = pltpu.async_copy(
                w_hbm.at[gp_v.at[pl.ds(c * 8, _SC_CHUNK)]],
                w_v.at[slot], sems[slot])
            return hx, hw

        pending = start_chunk(0)
        hb.wait()
        acc16 = jnp.zeros((_LANES,), jnp.float32)
        for c in range(n_chunks):
            slot = c % 2
            pending[0].wait()
            pending[1].wait()
            if c + 1 < n_chunks:
                pending = start_chunk(c + 1)
            for t in range(_SC_CHUNK):
                ti = c * _SC_CHUNK + t

                z = jnp.zeros((_LANES,), jnp.float32)

                @plsc.parallel_loop(0, steps, unroll=4, carry=(z, z, z, z))
                def accs(j, carry):
                    a0, a1, a2, a3 = carry
                    o = j * (_LANES * _UNROLL)
                    for k in range(_UNROLL):
                        off = o + k * _LANES
                        prod = (x_v[slot, t, pl.ds(off, _LANES)]
                                * w_v[slot, t, pl.ds(off, _LANES)])
                        if k % 4 == 0:
                            a0 = a0 + prod
                        elif k % 4 == 1:
                            a1 = a1 + prod
                        elif k % 4 == 2:
                            a2 = a2 + prod
                        else:
                            a3 = a3 + prod
                    return (a0, a1, a2, a3)

                v = (accs[0] + accs[1]) + (accs[2] + accs[3])
                for sh in (8, 4, 2, 1):
                    rot_v[pl.ds(0, _LANES)] = v
                    rot_v[pl.ds(_LANES, _LANES)] = v
                    v = v + rot_v[pl.ds(sh, _LANES)]
                acc16 = acc16 + jnp.where(lane == ti % _LANES, v, 0.0)
                if ti % _LANES == _LANES - 1:
                    blk = (ti // _LANES) * _LANES
                    y_v[pl.ds(blk, _LANES)] = (
                        acc16 + b_v[pl.ds(blk, _LANES)])
                    acc16 = jnp.zeros((_LANES,), jnp.float32)
        pltpu.sync_copy(y_v, out_hbm.at[pl.ds(base, _T_W)])

    return dot_k(x, group_sc, group_pad, w2, b_flat)


def kernel(x, group, W, b):
    g1 = group.astype(jnp.int32)
    w2 = W.reshape(NUM_GROUPS, D_MODEL)
    b2 = b.reshape(1, NUM_GROUPS)
    g_sc = g1[_TC_TOKENS:]
    g_pad = jnp.zeros((SC_TOKENS // _SC_CHUNK, 8), jnp.int32)
    g_pad = g_pad.at[:, :_SC_CHUNK].set(g_sc.reshape(-1, _SC_CHUNK))
    y_tc = _tc_part(x, g1.reshape(-1, 1), w2, b2)
    y_sc = _sc_part(x, g_sc, g_pad.reshape(-1), w2, b.reshape(-1))
    return jnp.concatenate([y_tc, y_sc.reshape(SC_TOKENS, 1)], axis=0)


# speedup vs baseline: 1.1737x; 1.0010x over previous
import functools

import jax
import jax.numpy as jnp
from jax import lax
from jax.experimental import pallas as pl
from jax.experimental.pallas import tpu as pltpu
from jax.experimental.pallas import tpu_sc as plsc

N_TOKENS = 8192
D_MODEL = 4096
NUM_GROUPS = 16
BLOCK_N = 512

_LANES = 16
_NUM_WORKERS = 32
SC_TOKENS = 1024
_TC_TOKENS = N_TOKENS - SC_TOKENS
_TOK0 = _TC_TOKENS
_T_W = SC_TOKENS // _NUM_WORKERS
_SC_CHUNK = 4
_UNROLL = 8


def _fused_kernel(x_ref, g_ref, w_ref, b_ref, o_ref):
    xb = x_ref[...]
    scores = lax.dot_general(
        xb, w_ref[...], (((1,), (1,)), ((), ())),
        preferred_element_type=jnp.float32)
    gid = g_ref[...]
    cols = lax.broadcasted_iota(jnp.int32, (xb.shape[0], NUM_GROUPS), 1)
    onehot = (cols == gid).astype(jnp.float32)
    o_ref[...] = jnp.sum((scores + b_ref[...]) * onehot, axis=1, keepdims=True)


def _tc_part(x_tc, g_tc, w2, b2):
    grid = _TC_TOKENS // BLOCK_N
    return pl.pallas_call(
        _fused_kernel,
        grid=(grid,),
        in_specs=[
            pl.BlockSpec((BLOCK_N, D_MODEL), lambda i: (i, 0)),
            pl.BlockSpec((BLOCK_N, 1), lambda i: (i, 0)),
            pl.BlockSpec((NUM_GROUPS, D_MODEL), lambda i: (0, 0)),
            pl.BlockSpec((1, NUM_GROUPS), lambda i: (0, 0)),
        ],
        out_specs=pl.BlockSpec((BLOCK_N, 1), lambda i: (i, 0)),
        out_shape=jax.ShapeDtypeStruct((_TC_TOKENS, 1), jnp.float32),
    )(x_tc, g_tc, w2, b2)


def _sc_part(x, group_sc, group_pad, w2, b_flat):
    mesh = plsc.VectorSubcoreMesh(core_axis_name="c", subcore_axis_name="s")
    steps = D_MODEL // (_LANES * _UNROLL)

    n_chunks = _T_W // _SC_CHUNK

    @functools.partial(
        pl.kernel, mesh=mesh,
        out_type=jax.ShapeDtypeStruct((SC_TOKENS,), jnp.float32),
        scratch_types=[
            pltpu.VMEM((2, _SC_CHUNK, D_MODEL), jnp.float32),
            pltpu.VMEM((2, _SC_CHUNK, D_MODEL), jnp.float32),
            pltpu.VMEM((8 * n_chunks,), jnp.int32),
            pltpu.VMEM((_T_W,), jnp.int32),
            pltpu.VMEM((_T_W,), jnp.float32),
            pltpu.VMEM((_T_W,), jnp.float32),
            pltpu.VMEM((2 * _LANES,), jnp.float32),
            pltpu.SemaphoreType.DMA,
            pltpu.SemaphoreType.DMA,
            pltpu.SemaphoreType.DMA,
        ],
    )
    def dot_k(x_hbm, g_hbm, gp_hbm, w_hbm, b_hbm, out_hbm,
              x_v, w_v, gp_v, g_v, b_v, y_v, rot_v, sem0, sem1, semb):
        wid = lax.axis_index("s") * 2 + lax.axis_index("c")
        base = wid * _T_W
        lane = lax.broadcasted_iota(jnp.int32, (_LANES,), 0)
        sems = (sem0, sem1)
        pltpu.sync_copy(gp_hbm.at[pl.ds(wid * 8 * n_chunks, 8 * n_chunks)],
                        gp_v)
        pltpu.sync_copy(g_hbm.at[pl.ds(base, _T_W)], g_v)
        hb = pltpu.async_copy(b_hbm.at[g_v], b_v, semb)

        def start_chunk(c):
            slot = c % 2
            tok = base + c * _SC_CHUNK
            hx = pltpu.async_copy(
                x_hbm.at[pl.ds(_TOK0 + tok, _SC_CHUNK)], x_v.at[slot],
                sems[slot])
            hw = pltpu.async_copy(
                w_hbm.at[gp_v.at[pl.ds(c * 8, _SC_CHUNK)]],
                w_v.at[slot], sems[slot])
            return hx, hw

        pending = start_chunk(0)
        hb.wait()
        acc16 = jnp.zeros((_LANES,), jnp.float32)
        for c in range(n_chunks):
            slot = c % 2
            pending[0].wait()
            pending[1].wait()
            if c + 1 < n_chunks:
                pending = start_chunk(c + 1)
            for t in range(_SC_CHUNK):
                ti = c * _SC_CHUNK + t

                z = jnp.zeros((_LANES,), jnp.float32)

                @plsc.parallel_loop(0, steps, unroll=4, carry=(z, z, z, z))
                def accs(j, carry):
                    a0, a1, a2, a3 = carry
                    o = j * (_LANES * _UNROLL)
                    for k in range(_UNROLL):
                        off = o + k * _LANES
                        prod = (x_v[slot, t, pl.ds(off, _LANES)]
                                * w_v[slot, t, pl.ds(off, _LANES)])
                        if k % 4 == 0:
                            a0 = a0 + prod
                        elif k % 4 == 1:
                            a1 = a1 + prod
                        elif k % 4 == 2:
                            a2 = a2 + prod
                        else:
                            a3 = a3 + prod
                    return (a0, a1, a2, a3)

                v = (accs[0] + accs[1]) + (accs[2] + accs[3])
                for sh in (8, 4, 2, 1):
                    rot_v[pl.ds(0, _LANES)] = v
                    rot_v[pl.ds(_LANES, _LANES)] = v
                    v = v + rot_v[pl.ds(sh, _LANES)]
                acc16 = acc16 + jnp.where(lane == ti % _LANES, v, 0.0)
                if ti % _LANES == _LANES - 1:
                    blk = (ti // _LANES) * _LANES
                    y_v[pl.ds(blk, _LANES)] = (
                        acc16 + b_v[pl.ds(blk, _LANES)])
                    acc16 = jnp.zeros((_LANES,), jnp.float32)
        pltpu.sync_copy(y_v, out_hbm.at[pl.ds(base, _T_W)])

    return dot_k(x, group_sc, group_pad, w2, b_flat)


def kernel(x, group, W, b):
    g1 = group.astype(jnp.int32)
    w2 = W.reshape(NUM_GROUPS, D_MODEL)
    b2 = b.reshape(1, NUM_GROUPS)
    g_sc = g1[_TC_TOKENS:]
    g_pad = jnp.zeros((SC_TOKENS // _SC_CHUNK, 8), jnp.int32)
    g_pad = g_pad.at[:, :_SC_CHUNK].set(g_sc.reshape(-1, _SC_CHUNK))
    y_sc = _sc_part(x, g_sc, g_pad.reshape(-1), w2, b.reshape(-1))
    y_tc = _tc_part(x, g1.reshape(-1, 1), w2, b2)
    return jnp.concatenate([y_tc, y_sc.reshape(SC_TOKENS, 1)], axis=0)


# speedup vs baseline: 1.5821x; 1.3480x over previous
import functools

import jax
import jax.numpy as jnp
from jax import lax
from jax.experimental import pallas as pl
from jax.experimental.pallas import tpu as pltpu
from jax.experimental.pallas import tpu_sc as plsc

N_TOKENS = 8192
D_MODEL = 4096
NUM_GROUPS = 16
BLOCK_N = 512
_HALF = BLOCK_N // 2

_LANES = 16
_NUM_WORKERS = 32
_TOK_PER_WORKER = N_TOKENS // _NUM_WORKERS


def _scores_kernel(xa_ref, xb_ref, w_ref, b_ref, o_ref):
    w = w_ref[...]
    dn = (((1,), (1,)), ((), ()))
    sa = lax.dot_general(xa_ref[...], w, dn,
                         preferred_element_type=jnp.float32)
    sb = lax.dot_general(xb_ref[...], w, dn,
                         preferred_element_type=jnp.float32)
    bias = b_ref[...]
    o_ref[:_HALF, :] = sa + bias
    o_ref[_HALF:, :] = sb + bias


def _tc_scores(x, w2, b2):
    grid = N_TOKENS // BLOCK_N
    return pl.pallas_call(
        _scores_kernel,
        grid=(grid,),
        in_specs=[
            pl.BlockSpec((_HALF, D_MODEL), lambda i: (2 * i, 0)),
            pl.BlockSpec((_HALF, D_MODEL), lambda i: (2 * i + 1, 0)),
            pl.BlockSpec((NUM_GROUPS, D_MODEL), lambda i: (0, 0)),
            pl.BlockSpec((1, NUM_GROUPS), lambda i: (0, 0)),
        ],
        out_specs=pl.BlockSpec((BLOCK_N, NUM_GROUPS), lambda i: (i, 0)),
        out_shape=jax.ShapeDtypeStruct((N_TOKENS, NUM_GROUPS), jnp.float32),
    )(x, x, w2, b2)


def _sc_select(scores, group):
    mesh = plsc.VectorSubcoreMesh(core_axis_name="c", subcore_axis_name="s")

    @functools.partial(
        pl.kernel, mesh=mesh,
        out_type=jax.ShapeDtypeStruct((N_TOKENS,), jnp.float32),
        scratch_types=[
            pltpu.VMEM((_TOK_PER_WORKER,), jnp.int32),
            pltpu.VMEM((_TOK_PER_WORKER,), jnp.int32),
            pltpu.VMEM((_TOK_PER_WORKER,), jnp.float32),
            pltpu.SemaphoreType.DMA,
        ],
    )
    def sel(scores_hbm, group_hbm, out_hbm, g_v, idx_v, y_v, sem):
        wid = lax.axis_index("s") * 2 + lax.axis_index("c")
        base = wid * _TOK_PER_WORKER
        pltpu.sync_copy(group_hbm.at[pl.ds(base, _TOK_PER_WORKER)], g_v)
        lane = lax.broadcasted_iota(jnp.int32, (_LANES,), 0)
        for i in range(_TOK_PER_WORKER // _LANES):
            cols = g_v[pl.ds(i * _LANES, _LANES)]
            idx_v[pl.ds(i * _LANES, _LANES)] = (
                (lane + (base + i * _LANES)) * NUM_GROUPS + cols)
        pltpu.async_copy(scores_hbm.at[idx_v], y_v, sem).wait()
        pltpu.sync_copy(y_v, out_hbm.at[pl.ds(base, _TOK_PER_WORKER)])

    return sel(scores.reshape(-1), group)


def kernel(x, group, W, b):
    g1 = group.astype(jnp.int32)
    w2 = W.reshape(NUM_GROUPS, D_MODEL)
    b2 = b.reshape(1, NUM_GROUPS)
    scores = _tc_scores(x, w2, b2)
    y = _sc_select(scores, g1)
    return y.reshape(N_TOKENS, 1)


# speedup vs baseline: 1.6901x; 1.0683x over previous
import functools

import jax
import jax.numpy as jnp
from jax import lax
from jax.experimental import pallas as pl
from jax.experimental.pallas import tpu as pltpu
from jax.experimental.pallas import tpu_sc as plsc

N_TOKENS = 8192
D_MODEL = 4096
NUM_GROUPS = 16
BLOCK_N = 512
_HALF = BLOCK_N // 2

_SCORE_STRIDE = 128

_LANES = 16
_NUM_WORKERS = 32
_TOK_PER_WORKER = N_TOKENS // _NUM_WORKERS


def _scores_kernel(xa_ref, xb_ref, w_ref, b_ref, o_ref):
    w = w_ref[...]
    dn = (((1,), (1,)), ((), ()))
    sa = lax.dot_general(xa_ref[...], w, dn,
                         preferred_element_type=jnp.float32)
    sb = lax.dot_general(xb_ref[...], w, dn,
                         preferred_element_type=jnp.float32)
    bias = b_ref[...]
    o_ref[:_HALF, :NUM_GROUPS] = sa + bias
    o_ref[_HALF:, :NUM_GROUPS] = sb + bias


def _tc_scores(x, w2, b2):
    grid = N_TOKENS // BLOCK_N
    return pl.pallas_call(
        _scores_kernel,
        grid=(grid,),
        in_specs=[
            pl.BlockSpec((_HALF, D_MODEL), lambda i: (2 * i, 0)),
            pl.BlockSpec((_HALF, D_MODEL), lambda i: (2 * i + 1, 0)),
            pl.BlockSpec((NUM_GROUPS, D_MODEL), lambda i: (0, 0)),
            pl.BlockSpec((1, NUM_GROUPS), lambda i: (0, 0)),
        ],
        out_specs=pl.BlockSpec((BLOCK_N, _SCORE_STRIDE), lambda i: (i, 0)),
        out_shape=jax.ShapeDtypeStruct((N_TOKENS, _SCORE_STRIDE), jnp.float32),
    )(x, x, w2, b2)


def _sc_select(scores, group):
    mesh = plsc.VectorSubcoreMesh(core_axis_name="c", subcore_axis_name="s")

    @functools.partial(
        pl.kernel, mesh=mesh,
        out_type=jax.ShapeDtypeStruct((N_TOKENS,), jnp.float32),
        scratch_types=[
            pltpu.VMEM((_TOK_PER_WORKER,), jnp.int32),
            pltpu.VMEM((_TOK_PER_WORKER,), jnp.int32),
            pltpu.VMEM((_TOK_PER_WORKER,), jnp.float32),
            pltpu.SemaphoreType.DMA,
        ],
    )
    def sel(scores_hbm, group_hbm, out_hbm, g_v, idx_v, y_v, sem):
        wid = lax.axis_index("s") * 2 + lax.axis_index("c")
        base = wid * _TOK_PER_WORKER
        pltpu.sync_copy(group_hbm.at[pl.ds(base, _TOK_PER_WORKER)], g_v)
        lane = lax.broadcasted_iota(jnp.int32, (_LANES,), 0)
        for i in range(_TOK_PER_WORKER // _LANES):
            cols = g_v[pl.ds(i * _LANES, _LANES)]
            idx_v[pl.ds(i * _LANES, _LANES)] = (
                (lane + (base + i * _LANES)) * _SCORE_STRIDE + cols)
        pltpu.async_copy(scores_hbm.at[idx_v], y_v, sem).wait()
        pltpu.sync_copy(y_v, out_hbm.at[pl.ds(base, _TOK_PER_WORKER)])

    return sel(scores.reshape(-1), group)


def kernel(x, group, W, b):
    g1 = group.astype(jnp.int32)
    w2 = W.reshape(NUM_GROUPS, D_MODEL)
    b2 = b.reshape(1, NUM_GROUPS)
    scores = _tc_scores(x, w2, b2)
    y = _sc_select(scores, g1)
    return y.reshape(N_TOKENS, 1)
